# Initial kernel scaffold; baseline (speedup 1.0000x reference)
import jax
import jax.numpy as jnp
from jax.experimental import pallas as pl


def kernel(x, Adj_, W1, b1, W2, b2, W3, b3, Wp1, bp1, Wp2, bp2):
    raise NotImplementedError("write your pallas kernel here")


# speedup vs baseline: 1.0120x; 1.0120x over previous
import jax
import jax.numpy as jnp
from jax.experimental import pallas as pl
from jax.experimental.pallas import tpu as pltpu


def _pick_block(n, target, mult):
    best = None
    for d in range(mult, target + 1, mult):
        if n % d == 0:
            best = d
    return best if best is not None else n


def _linear(x, w, b):
    n, d = x.shape
    h = w.shape[1]
    bm = _pick_block(n, 2000, 8)

    def body(x_ref, w_ref, b_ref, o_ref):
        o_ref[...] = (
            jnp.dot(x_ref[...], w_ref[...], preferred_element_type=jnp.float32)
            + b_ref[...]
        ).astype(jnp.bfloat16)

    return pl.pallas_call(
        body,
        grid=(n // bm,),
        in_specs=[
            pl.BlockSpec((bm, d), lambda i: (i, 0)),
            pl.BlockSpec((d, h), lambda i: (0, 0)),
            pl.BlockSpec((1, h), lambda i: (0, 0)),
        ],
        out_specs=pl.BlockSpec((bm, h), lambda i: (i, 0)),
        out_shape=jax.ShapeDtypeStruct((n, h), jnp.bfloat16),
        compiler_params=pltpu.CompilerParams(
            dimension_semantics=("parallel",)
        ),
    )(x, w, b)


def _prop_first(adj, v, w, b, bm=80):
    n = adj.shape[0]
    h = v.shape[1]

    def body(adj_ref, v_ref, w_ref, b_ref, adjbf_ref, o_ref):
        a = adj_ref[...].astype(jnp.bfloat16)
        adjbf_ref[...] = a
        hh = jnp.maximum(
            jnp.dot(a, v_ref[...], preferred_element_type=jnp.float32), 0.0
        )
        o_ref[...] = (
            jnp.dot(hh, w_ref[...], preferred_element_type=jnp.float32)
            + b_ref[...]
        ).astype(jnp.bfloat16)

    return pl.pallas_call(
        body,
        grid=(n // bm,),
        in_specs=[
            pl.BlockSpec((bm, n), lambda i: (i, 0)),
            pl.BlockSpec((n, h), lambda i: (0, 0)),
            pl.BlockSpec((h, h), lambda i: (0, 0)),
            pl.BlockSpec((1, h), lambda i: (0, 0)),
        ],
        out_specs=[
            pl.BlockSpec((bm, n), lambda i: (i, 0)),
            pl.BlockSpec((bm, h), lambda i: (i, 0)),
        ],
        out_shape=[
            jax.ShapeDtypeStruct((n, n), jnp.bfloat16),
            jax.ShapeDtypeStruct((n, h), jnp.bfloat16),
        ],
        compiler_params=pltpu.CompilerParams(
            dimension_semantics=("parallel",)
        ),
    )(adj, v, w, b)


def _prop_mid(adj_bf, v, w, b, bm=400):
    n = adj_bf.shape[0]
    h = v.shape[1]

    def body(adj_ref, v_ref, w_ref, b_ref, o_ref):
        hh = jnp.maximum(
            jnp.dot(adj_ref[...], v_ref[...],
                    preferred_element_type=jnp.float32),
            0.0,
        )
        o_ref[...] = (
            jnp.dot(hh, w_ref[...], preferred_element_type=jnp.float32)
            + b_ref[...]
        ).astype(jnp.bfloat16)

    return pl.pallas_call(
        body,
        grid=(n // bm,),
        in_specs=[
            pl.BlockSpec((bm, n), lambda i: (i, 0)),
            pl.BlockSpec((n, h), lambda i: (0, 0)),
            pl.BlockSpec((h, h), lambda i: (0, 0)),
            pl.BlockSpec((1, h), lambda i: (0, 0)),
        ],
        out_specs=pl.BlockSpec((bm, h), lambda i: (i, 0)),
        out_shape=jax.ShapeDtypeStruct((n, h), jnp.bfloat16),
        compiler_params=pltpu.CompilerParams(
            dimension_semantics=("parallel",)
        ),
    )(adj_bf, v, w, b)


def _prop_last(adj_bf, v, wp1, bp1, wp2, bp2, bm=400):
    n = adj_bf.shape[0]
    h = v.shape[1]
    p = wp1.shape[1]
    p2 = wp2.shape[1]

    def body(adj_ref, v_ref, wp1_ref, bp1_ref, wp2_ref, bp2_ref,
             emb_ref, z_ref):
        emb = jnp.dot(adj_ref[...], v_ref[...],
                      preferred_element_type=jnp.float32)
        emb_ref[...] = emb
        t = jnp.maximum(
            jnp.dot(emb, wp1_ref[...], preferred_element_type=jnp.float32)
            + bp1_ref[...],
            0.0,
        )
        z_ref[...] = (
            jnp.dot(t, wp2_ref[...], preferred_element_type=jnp.float32)
            + bp2_ref[...]
        )

    return pl.pallas_call(
        body,
        grid=(n // bm,),
        in_specs=[
            pl.BlockSpec((bm, n), lambda i: (i, 0)),
            pl.BlockSpec((n, h), lambda i: (0, 0)),
            pl.BlockSpec((h, p), lambda i: (0, 0)),
            pl.BlockSpec((1, p), lambda i: (0, 0)),
            pl.BlockSpec((p, p2), lambda i: (0, 0)),
            pl.BlockSpec((1, p2), lambda i: (0, 0)),
        ],
        out_specs=[
            pl.BlockSpec((bm, h), lambda i: (i, 0)),
            pl.BlockSpec((bm, p2), lambda i: (i, 0)),
        ],
        out_shape=[
            jax.ShapeDtypeStruct((n, h), jnp.float32),
            jax.ShapeDtypeStruct((n, p2), jnp.float32),
        ],
        compiler_params=pltpu.CompilerParams(
            dimension_semantics=("parallel",)
        ),
    )(adj_bf, v, wp1, bp1, wp2, bp2)


def kernel(x, Adj_, W1, b1, W2, b2, W3, b3, Wp1, bp1, Wp2, bp2):
    n = Adj_.shape[0]
    bm1 = _pick_block(n, 80, 16)
    bm2 = _pick_block(n, 400, 16)
    v1 = _linear(x, W1, b1.reshape(1, -1))
    adj_bf, v2 = _prop_first(Adj_, v1, W2, b2.reshape(1, -1), bm=bm1)
    v3 = _prop_mid(adj_bf, v2, W3, b3.reshape(1, -1), bm=bm2)
    emb, z = _prop_last(
        adj_bf, v3, Wp1, bp1.reshape(1, -1), Wp2, bp2.reshape(1, -1), bm=bm2
    )
    return (z, emb)


# speedup vs baseline: 1.1053x; 1.0921x over previous
import jax
import jax.numpy as jnp
from jax.experimental import pallas as pl
from jax.experimental.pallas import tpu as pltpu


def _pick_block(n, target, mult):
    best = None
    for d in range(mult, target + 1, mult):
        if n % d == 0:
            best = d
    return best if best is not None else n


def _linear(x, w, b):
    n, d = x.shape
    h = w.shape[1]
    bm = _pick_block(n, 2000, 8)

    def body(x_ref, w_ref, b_ref, o_ref):
        o_ref[...] = (
            jnp.dot(x_ref[...], w_ref[...], preferred_element_type=jnp.float32)
            + b_ref[...]
        ).astype(jnp.bfloat16)

    return pl.pallas_call(
        body,
        grid=(n // bm,),
        in_specs=[
            pl.BlockSpec((bm, d), lambda i: (i, 0)),
            pl.BlockSpec((d, h), lambda i: (0, 0)),
            pl.BlockSpec((1, h), lambda i: (0, 0)),
        ],
        out_specs=pl.BlockSpec((bm, h), lambda i: (i, 0)),
        out_shape=jax.ShapeDtypeStruct((n, h), jnp.bfloat16),
        compiler_params=pltpu.CompilerParams(
            dimension_semantics=("parallel",)
        ),
    )(x, w, b)


def _prop_first(adj, v, w, b, bm=80):
    n = adj.shape[0]
    h = v.shape[1]

    def body(adj_ref, v_ref, w_ref, b_ref, adjbf_ref, o_ref):
        a = adj_ref[...].astype(jnp.bfloat16)
        adjbf_ref[...] = a
        hh = jnp.maximum(
            jnp.dot(a, v_ref[...], preferred_element_type=jnp.float32), 0.0
        )
        o_ref[...] = (
            jnp.dot(hh, w_ref[...], preferred_element_type=jnp.float32)
            + b_ref[...]
        ).astype(jnp.bfloat16)

    return pl.pallas_call(
        body,
        grid=(n // bm,),
        in_specs=[
            pl.BlockSpec((bm, n), lambda i: (i, 0)),
            pl.BlockSpec((n, h), lambda i: (0, 0)),
            pl.BlockSpec((h, h), lambda i: (0, 0)),
            pl.BlockSpec((1, h), lambda i: (0, 0)),
        ],
        out_specs=[
            pl.BlockSpec((bm, n), lambda i: (i, 0)),
            pl.BlockSpec((bm, h), lambda i: (i, 0)),
        ],
        out_shape=[
            jax.ShapeDtypeStruct((n, n), jnp.bfloat16),
            jax.ShapeDtypeStruct((n, h), jnp.bfloat16),
        ],
        compiler_params=pltpu.CompilerParams(
            dimension_semantics=("parallel",)
        ),
    )(adj, v, w, b)


def _prop_mid(adj_bf, v, w, b, bm=400):
    n = adj_bf.shape[0]
    h = v.shape[1]

    def body(adj_ref, v_ref, w_ref, b_ref, o_ref):
        hh = jnp.maximum(
            jnp.dot(adj_ref[...], v_ref[...],
                    preferred_element_type=jnp.float32),
            0.0,
        )
        o_ref[...] = (
            jnp.dot(hh, w_ref[...], preferred_element_type=jnp.float32)
            + b_ref[...]
        ).astype(jnp.bfloat16)

    return pl.pallas_call(
        body,
        grid=(n // bm,),
        in_specs=[
            pl.BlockSpec((bm, n), lambda i: (i, 0)),
            pl.BlockSpec((n, h), lambda i: (0, 0)),
            pl.BlockSpec((h, h), lambda i: (0, 0)),
            pl.BlockSpec((1, h), lambda i: (0, 0)),
        ],
        out_specs=pl.BlockSpec((bm, h), lambda i: (i, 0)),
        out_shape=jax.ShapeDtypeStruct((n, h), jnp.bfloat16),
        compiler_params=pltpu.CompilerParams(
            dimension_semantics=("parallel",)
        ),
    )(adj_bf, v, w, b)


def _prop_last(adj_bf, v, wp1, bp1, wp2, bp2, bm=400):
    n = adj_bf.shape[0]
    h = v.shape[1]
    p = wp1.shape[1]
    p2 = wp2.shape[1]

    def body(adj_ref, v_ref, wp1_ref, bp1_ref, wp2_ref, bp2_ref,
             emb_ref, z_ref):
        emb = jnp.dot(adj_ref[...], v_ref[...],
                      preferred_element_type=jnp.float32)
        emb_ref[...] = emb
        t = jnp.maximum(
            jnp.dot(emb, wp1_ref[...], preferred_element_type=jnp.float32)
            + bp1_ref[...],
            0.0,
        )
        z_ref[...] = (
            jnp.dot(t, wp2_ref[...], preferred_element_type=jnp.float32)
            + bp2_ref[...]
        )

    return pl.pallas_call(
        body,
        grid=(n // bm,),
        in_specs=[
            pl.BlockSpec((bm, n), lambda i: (i, 0)),
            pl.BlockSpec((n, h), lambda i: (0, 0)),
            pl.BlockSpec((h, p), lambda i: (0, 0)),
            pl.BlockSpec((1, p), lambda i: (0, 0)),
            pl.BlockSpec((p, p2), lambda i: (0, 0)),
            pl.BlockSpec((1, p2), lambda i: (0, 0)),
        ],
        out_specs=[
            pl.BlockSpec((bm, h), lambda i: (i, 0)),
            pl.BlockSpec((bm, p2), lambda i: (i, 0)),
        ],
        out_shape=[
            jax.ShapeDtypeStruct((n, h), jnp.float32),
            jax.ShapeDtypeStruct((n, p2), jnp.float32),
        ],
        compiler_params=pltpu.CompilerParams(
            dimension_semantics=("parallel",)
        ),
    )(adj_bf, v, wp1, bp1, wp2, bp2)


def kernel(x, Adj_, W1, b1, W2, b2, W3, b3, Wp1, bp1, Wp2, bp2):
    n = Adj_.shape[0]
    bm1 = _pick_block(n, 400, 16)
    bm2 = _pick_block(n, 400, 16)
    v1 = _linear(x, W1, b1.reshape(1, -1))
    adj_bf, v2 = _prop_first(Adj_, v1, W2, b2.reshape(1, -1), bm=bm1)
    v3 = _prop_mid(adj_bf, v2, W3, b3.reshape(1, -1), bm=bm2)
    emb, z = _prop_last(
        adj_bf, v3, Wp1, bp1.reshape(1, -1), Wp2, bp2.reshape(1, -1), bm=bm2
    )
    return (z, emb)


# speedup vs baseline: 1.1522x; 1.0425x over previous
import jax
import jax.numpy as jnp
from jax.experimental import pallas as pl
from jax.experimental.pallas import tpu as pltpu


def _pick_block(n, target, mult):
    best = None
    for d in range(mult, target + 1, mult):
        if n % d == 0:
            best = d
    return best if best is not None else n


def _linear(x, w, b):
    n, d = x.shape
    h = w.shape[1]
    bm = _pick_block(n, 2000, 8)

    def body(x_ref, w_ref, b_ref, o_ref):
        o_ref[...] = (
            jnp.dot(x_ref[...], w_ref[...], preferred_element_type=jnp.float32)
            + b_ref[...]
        ).astype(jnp.bfloat16)

    return pl.pallas_call(
        body,
        grid=(n // bm,),
        in_specs=[
            pl.BlockSpec((bm, d), lambda i: (i, 0)),
            pl.BlockSpec((d, h), lambda i: (0, 0)),
            pl.BlockSpec((1, h), lambda i: (0, 0)),
        ],
        out_specs=pl.BlockSpec((bm, h), lambda i: (i, 0)),
        out_shape=jax.ShapeDtypeStruct((n, h), jnp.bfloat16),
        compiler_params=pltpu.CompilerParams(
            dimension_semantics=("parallel",)
        ),
    )(x, w, b)


def _prop_first(adj, v, w, b, bm=80):
    n = adj.shape[0]
    h = v.shape[1]

    def body(adj_ref, v_ref, w_ref, b_ref, adjbf_ref, o_ref):
        a = adj_ref[...].astype(jnp.bfloat16)
        adjbf_ref[...] = a
        hh = jnp.maximum(
            jnp.dot(a, v_ref[...], preferred_element_type=jnp.float32), 0.0
        )
        o_ref[...] = (
            jnp.dot(hh, w_ref[...], preferred_element_type=jnp.float32)
            + b_ref[...]
        ).astype(jnp.bfloat16)

    return pl.pallas_call(
        body,
        grid=(n // bm,),
        in_specs=[
            pl.BlockSpec((bm, n), lambda i: (i, 0)),
            pl.BlockSpec((n, h), lambda i: (0, 0)),
            pl.BlockSpec((h, h), lambda i: (0, 0)),
            pl.BlockSpec((1, h), lambda i: (0, 0)),
        ],
        out_specs=[
            pl.BlockSpec((bm, n), lambda i: (i, 0)),
            pl.BlockSpec((bm, h), lambda i: (i, 0)),
        ],
        out_shape=[
            jax.ShapeDtypeStruct((n, n), jnp.bfloat16),
            jax.ShapeDtypeStruct((n, h), jnp.bfloat16),
        ],
        compiler_params=pltpu.CompilerParams(
            dimension_semantics=("parallel",)
        ),
    )(adj, v, w, b)


def _prop_mid(adj_bf, v, w, b, bm=400):
    n = adj_bf.shape[0]
    h = v.shape[1]

    def body(adj_ref, v_ref, w_ref, b_ref, o_ref):
        hh = jnp.maximum(
            jnp.dot(adj_ref[...], v_ref[...],
                    preferred_element_type=jnp.float32),
            0.0,
        )
        o_ref[...] = (
            jnp.dot(hh, w_ref[...], preferred_element_type=jnp.float32)
            + b_ref[...]
        ).astype(jnp.bfloat16)

    return pl.pallas_call(
        body,
        grid=(n // bm,),
        in_specs=[
            pl.BlockSpec((bm, n), lambda i: (i, 0)),
            pl.BlockSpec((n, h), lambda i: (0, 0)),
            pl.BlockSpec((h, h), lambda i: (0, 0)),
            pl.BlockSpec((1, h), lambda i: (0, 0)),
        ],
        out_specs=pl.BlockSpec((bm, h), lambda i: (i, 0)),
        out_shape=jax.ShapeDtypeStruct((n, h), jnp.bfloat16),
        compiler_params=pltpu.CompilerParams(
            dimension_semantics=("parallel",)
        ),
    )(adj_bf, v, w, b)


def _prop_last(adj_bf, v, wp1, bp1, wp2, bp2, bm=400):
    n = adj_bf.shape[0]
    h = v.shape[1]
    p = wp1.shape[1]
    p2 = wp2.shape[1]

    def body(adj_ref, v_ref, wp1_ref, bp1_ref, wp2_ref, bp2_ref,
             emb_ref, z_ref):
        emb = jnp.dot(adj_ref[...], v_ref[...],
                      preferred_element_type=jnp.float32)
        emb_ref[...] = emb
        t = jnp.maximum(
            jnp.dot(emb, wp1_ref[...], preferred_element_type=jnp.float32)
            + bp1_ref[...],
            0.0,
        )
        z_ref[...] = (
            jnp.dot(t, wp2_ref[...], preferred_element_type=jnp.float32)
            + bp2_ref[...]
        )

    return pl.pallas_call(
        body,
        grid=(n // bm,),
        in_specs=[
            pl.BlockSpec((bm, n), lambda i: (i, 0)),
            pl.BlockSpec((n, h), lambda i: (0, 0)),
            pl.BlockSpec((h, p), lambda i: (0, 0)),
            pl.BlockSpec((1, p), lambda i: (0, 0)),
            pl.BlockSpec((p, p2), lambda i: (0, 0)),
            pl.BlockSpec((1, p2), lambda i: (0, 0)),
        ],
        out_specs=[
            pl.BlockSpec((bm, h), lambda i: (i, 0)),
            pl.BlockSpec((bm, p2), lambda i: (i, 0)),
        ],
        out_shape=[
            jax.ShapeDtypeStruct((n, h), jnp.float32),
            jax.ShapeDtypeStruct((n, p2), jnp.float32),
        ],
        compiler_params=pltpu.CompilerParams(
            dimension_semantics=("parallel",)
        ),
    )(adj_bf, v, wp1, bp1, wp2, bp2)


def kernel(x, Adj_, W1, b1, W2, b2, W3, b3, Wp1, bp1, Wp2, bp2):
    n = Adj_.shape[0]
    bm1 = _pick_block(n, 400, 16)
    bm2 = _pick_block(n, 1000, 8)
    v1 = _linear(x, W1, b1.reshape(1, -1))
    adj_bf, v2 = _prop_first(Adj_, v1, W2, b2.reshape(1, -1), bm=bm1)
    v3 = _prop_mid(adj_bf, v2, W3, b3.reshape(1, -1), bm=bm2)
    emb, z = _prop_last(
        adj_bf, v3, Wp1, bp1.reshape(1, -1), Wp2, bp2.reshape(1, -1), bm=bm2
    )
    return (z, emb)


# speedup vs baseline: 1.2710x; 1.1031x over previous
import jax
import jax.numpy as jnp
from jax.experimental import pallas as pl
from jax.experimental.pallas import tpu as pltpu

_F8 = jnp.float8_e4m3fn


def _linear_quant(x, w, b):
    n, d = x.shape
    h = w.shape[1]

    def body(x_ref, w_ref, b_ref, qhi_ref, qlo_ref, s_ref, c_ref):
        v = (
            jnp.dot(x_ref[...], w_ref[...], preferred_element_type=jnp.float32)
            + b_ref[...]
        )
        m = jnp.maximum(jnp.max(jnp.abs(v)), 1e-30)
        vs = v * (256.0 / m)
        hi = vs.astype(_F8)
        qhi_ref[...] = hi
        qlo_ref[...] = ((vs - hi.astype(jnp.float32)) * 32.0).astype(_F8)
        s_ref[...] = jnp.full((1, 1), m / 256.0, jnp.float32)
        c_ref[...] = 0.5 * jnp.sum(v, axis=0, keepdims=True)

    return pl.pallas_call(
        body,
        grid=(1,),
        in_specs=[
            pl.BlockSpec((n, d), lambda i: (0, 0)),
            pl.BlockSpec((d, h), lambda i: (0, 0)),
            pl.BlockSpec((1, h), lambda i: (0, 0)),
        ],
        out_specs=[
            pl.BlockSpec((n, h), lambda i: (0, 0)),
            pl.BlockSpec((n, h), lambda i: (0, 0)),
            pl.BlockSpec((1, 1), lambda i: (0, 0)),
            pl.BlockSpec((1, h), lambda i: (0, 0)),
        ],
        out_shape=[
            jax.ShapeDtypeStruct((n, h), _F8),
            jax.ShapeDtypeStruct((n, h), _F8),
            jax.ShapeDtypeStruct((1, 1), jnp.float32),
            jax.ShapeDtypeStruct((1, h), jnp.float32),
        ],
    )(x, w, b)


def _quant(v):
    n, h = v.shape

    def body(v_ref, qhi_ref, qlo_ref, s_ref, c_ref):
        m = jnp.maximum(jnp.max(jnp.abs(v_ref[...])), 1e-30)
        vs = v_ref[...] * (256.0 / m)
        hi = vs.astype(_F8)
        qhi_ref[...] = hi
        qlo_ref[...] = ((vs - hi.astype(jnp.float32)) * 32.0).astype(_F8)
        s_ref[...] = jnp.full((1, 1), m / 256.0, jnp.float32)
        c_ref[...] = 0.5 * jnp.sum(v_ref[...], axis=0, keepdims=True)

    return pl.pallas_call(
        body,
        grid=(1,),
        in_specs=[pl.BlockSpec((n, h), lambda i: (0, 0))],
        out_specs=[
            pl.BlockSpec((n, h), lambda i: (0, 0)),
            pl.BlockSpec((n, h), lambda i: (0, 0)),
            pl.BlockSpec((1, 1), lambda i: (0, 0)),
            pl.BlockSpec((1, h), lambda i: (0, 0)),
        ],
        out_shape=[
            jax.ShapeDtypeStruct((n, h), _F8),
            jax.ShapeDtypeStruct((n, h), _F8),
            jax.ShapeDtypeStruct((1, 1), jnp.float32),
            jax.ShapeDtypeStruct((1, h), jnp.float32),
        ],
    )(v)


def _prop_first(adj, vhi, vlo, s, c, w, b, bm=480):
    n = adj.shape[0]
    h = vhi.shape[1]
    grid = (n + bm - 1) // bm

    def body(adj_ref, vhi_ref, vlo_ref, s_ref, c_ref, w_ref, b_ref,
             adjq_ref, o_ref):
        a = (adj_ref[...] - 0.5).astype(_F8)
        adjq_ref[...] = a
        acc = (
            jnp.dot(a, vhi_ref[...], preferred_element_type=jnp.float32)
            + jnp.dot(a, vlo_ref[...], preferred_element_type=jnp.float32)
            * (1.0 / 32.0)
        )
        hh = jnp.maximum(acc * s_ref[0, 0] + c_ref[...], 0.0)
        o_ref[...] = (
            jnp.dot(hh, w_ref[...], preferred_element_type=jnp.float32)
            + b_ref[...]
        )

    return pl.pallas_call(
        body,
        grid=(grid,),
        in_specs=[
            pl.BlockSpec((bm, n), lambda i: (i, 0)),
            pl.BlockSpec((n, h), lambda i: (0, 0)),
            pl.BlockSpec((n, h), lambda i: (0, 0)),
            pl.BlockSpec((1, 1), lambda i: (0, 0)),
            pl.BlockSpec((1, h), lambda i: (0, 0)),
            pl.BlockSpec((h, h), lambda i: (0, 0)),
            pl.BlockSpec((1, h), lambda i: (0, 0)),
        ],
        out_specs=[
            pl.BlockSpec((bm, n), lambda i: (i, 0)),
            pl.BlockSpec((bm, h), lambda i: (i, 0)),
        ],
        out_shape=[
            jax.ShapeDtypeStruct((n, n), _F8),
            jax.ShapeDtypeStruct((n, h), jnp.float32),
        ],
        compiler_params=pltpu.CompilerParams(
            dimension_semantics=("parallel",)
        ),
    )(adj, vhi, vlo, s, c, w, b)


def _prop_mid(adj_q, vhi, vlo, s, c, w, b, bm=1024):
    n = adj_q.shape[0]
    h = vhi.shape[1]
    grid = (n + bm - 1) // bm

    def body(adj_ref, vhi_ref, vlo_ref, s_ref, c_ref, w_ref, b_ref, o_ref):
        a = adj_ref[...]
        acc = (
            jnp.dot(a, vhi_ref[...], preferred_element_type=jnp.float32)
            + jnp.dot(a, vlo_ref[...], preferred_element_type=jnp.float32)
            * (1.0 / 32.0)
        )
        hh = jnp.maximum(acc * s_ref[0, 0] + c_ref[...], 0.0)
        o_ref[...] = (
            jnp.dot(hh, w_ref[...], preferred_element_type=jnp.float32)
            + b_ref[...]
        )

    return pl.pallas_call(
        body,
        grid=(grid,),
        in_specs=[
            pl.BlockSpec((bm, n), lambda i: (i, 0)),
            pl.BlockSpec((n, h), lambda i: (0, 0)),
            pl.BlockSpec((n, h), lambda i: (0, 0)),
            pl.BlockSpec((1, 1), lambda i: (0, 0)),
            pl.BlockSpec((1, h), lambda i: (0, 0)),
            pl.BlockSpec((h, h), lambda i: (0, 0)),
            pl.BlockSpec((1, h), lambda i: (0, 0)),
        ],
        out_specs=pl.BlockSpec((bm, h), lambda i: (i, 0)),
        out_shape=jax.ShapeDtypeStruct((n, h), jnp.float32),
        compiler_params=pltpu.CompilerParams(
            dimension_semantics=("parallel",)
        ),
    )(adj_q, vhi, vlo, s, c, w, b)


def _prop_last(adj_q, vhi, vlo, s, c, wp1, bp1, wp2, bp2, bm=1024):
    n = adj_q.shape[0]
    h = vhi.shape[1]
    p = wp1.shape[1]
    p2 = wp2.shape[1]
    grid = (n + bm - 1) // bm

    def body(adj_ref, vhi_ref, vlo_ref, s_ref, c_ref, wp1_ref, bp1_ref,
             wp2_ref, bp2_ref, emb_ref, z_ref):
        a = adj_ref[...]
        acc = (
            jnp.dot(a, vhi_ref[...], preferred_element_type=jnp.float32)
            + jnp.dot(a, vlo_ref[...], preferred_element_type=jnp.float32)
            * (1.0 / 32.0)
        )
        emb = acc * s_ref[0, 0] + c_ref[...]
        emb_ref[...] = emb
        t = jnp.maximum(
            jnp.dot(emb, wp1_ref[...], preferred_element_type=jnp.float32)
            + bp1_ref[...],
            0.0,
        )
        z_ref[...] = (
            jnp.dot(t, wp2_ref[...], preferred_element_type=jnp.float32)
            + bp2_ref[...]
        )

    return pl.pallas_call(
        body,
        grid=(grid,),
        in_specs=[
            pl.BlockSpec((bm, n), lambda i: (i, 0)),
            pl.BlockSpec((n, h), lambda i: (0, 0)),
            pl.BlockSpec((n, h), lambda i: (0, 0)),
            pl.BlockSpec((1, 1), lambda i: (0, 0)),
            pl.BlockSpec((1, h), lambda i: (0, 0)),
            pl.BlockSpec((h, p), lambda i: (0, 0)),
            pl.BlockSpec((1, p), lambda i: (0, 0)),
            pl.BlockSpec((p, p2), lambda i: (0, 0)),
            pl.BlockSpec((1, p2), lambda i: (0, 0)),
        ],
        out_specs=[
            pl.BlockSpec((bm, h), lambda i: (i, 0)),
            pl.BlockSpec((bm, p2), lambda i: (i, 0)),
        ],
        out_shape=[
            jax.ShapeDtypeStruct((n, h), jnp.float32),
            jax.ShapeDtypeStruct((n, p2), jnp.float32),
        ],
        compiler_params=pltpu.CompilerParams(
            dimension_semantics=("parallel",)
        ),
    )(adj_q, vhi, vlo, s, c, wp1, bp1, wp2, bp2)


def kernel(x, Adj_, W1, b1, W2, b2, W3, b3, Wp1, bp1, Wp2, bp2):
    v1h, v1l, s1, c1 = _linear_quant(x, W1, b1.reshape(1, -1))
    adj_q, v2 = _prop_first(Adj_, v1h, v1l, s1, c1, W2, b2.reshape(1, -1))
    v2h, v2l, s2, c2 = _quant(v2)
    v3 = _prop_mid(adj_q, v2h, v2l, s2, c2, W3, b3.reshape(1, -1))
    v3h, v3l, s3, c3 = _quant(v3)
    emb, z = _prop_last(
        adj_q, v3h, v3l, s3, c3,
        Wp1, bp1.reshape(1, -1), Wp2, bp2.reshape(1, -1)
    )
    return (z, emb)


# speedup vs baseline: 1.5151x; 1.1921x over previous
import jax
import jax.numpy as jnp
from jax.experimental import pallas as pl
from jax.experimental.pallas import tpu as pltpu

_F8 = jnp.float8_e4m3fn


_QCH = 400


def _quantize_to_scratch(get_chunk, n, h, vq_ref, s_ref, c_ref):
    m = jnp.float32(1e-30)
    csum = jnp.zeros((1, h), jnp.float32)
    for j in range(0, n, _QCH):
        vv = get_chunk(j)
        m = jnp.maximum(m, jnp.max(jnp.abs(vv)))
        csum = csum + jnp.sum(vv, axis=0, keepdims=True)
    f = 256.0 / m
    for j in range(0, n, _QCH):
        vs = get_chunk(j) * f
        hi = vs.astype(_F8)
        vq_ref[j:j + _QCH, :h] = hi
        vq_ref[j:j + _QCH, h:] = ((vs - hi.astype(jnp.float32))
                                  * 32.0).astype(_F8)
    s_ref[...] = jnp.full((1, 1), m / 256.0, jnp.float32)
    c_ref[...] = 0.5 * csum


def _dequant_dot(a_ref, vq_ref, s_ref, c_ref, h):
    acc2 = jnp.dot(a_ref[...], vq_ref[...],
                   preferred_element_type=jnp.float32)
    acc = acc2[:, :h] + acc2[:, h:] * (1.0 / 32.0)
    return acc * s_ref[0, 0] + c_ref[...]


def _prop_first(adj, x, w1, b1, w2, b2, bm=448):
    n = adj.shape[0]
    d = x.shape[1]
    h = w2.shape[1]
    grid = (n + bm - 1) // bm

    def body(adj_ref, x_ref, w1_ref, b1_ref, w2_ref, b2_ref,
             adjq_ref, o_ref, vq_ref, s_ref, c_ref):
        @pl.when(pl.program_id(0) == 0)
        def _():
            def v1_chunk(j):
                return (
                    jnp.dot(x_ref[j:j + _QCH, :], w1_ref[...],
                            preferred_element_type=jnp.float32)
                    + b1_ref[...]
                )
            _quantize_to_scratch(v1_chunk, n, h, vq_ref, s_ref, c_ref)

        adjq_ref[...] = (adj_ref[...] - 0.5).astype(_F8)
        hh = jnp.maximum(_dequant_dot(adjq_ref, vq_ref, s_ref, c_ref, h), 0.0)
        o_ref[...] = (
            jnp.dot(hh, w2_ref[...], preferred_element_type=jnp.float32)
            + b2_ref[...]
        )

    return pl.pallas_call(
        body,
        grid=(grid,),
        in_specs=[
            pl.BlockSpec((bm, n), lambda i: (i, 0)),
            pl.BlockSpec((n, d), lambda i: (0, 0)),
            pl.BlockSpec((d, h), lambda i: (0, 0)),
            pl.BlockSpec((1, h), lambda i: (0, 0)),
            pl.BlockSpec((h, h), lambda i: (0, 0)),
            pl.BlockSpec((1, h), lambda i: (0, 0)),
        ],
        out_specs=[
            pl.BlockSpec((bm, n), lambda i: (i, 0)),
            pl.BlockSpec((bm, h), lambda i: (i, 0)),
        ],
        out_shape=[
            jax.ShapeDtypeStruct((n, n), _F8),
            jax.ShapeDtypeStruct((n, h), jnp.float32),
        ],
        scratch_shapes=[
            pltpu.VMEM((n, 2 * h), _F8),
            pltpu.VMEM((1, 1), jnp.float32),
            pltpu.VMEM((1, h), jnp.float32),
        ],
        compiler_params=pltpu.CompilerParams(
            dimension_semantics=("arbitrary",)
        ),
    )(adj, x, w1, b1, w2, b2)


def _prop_mid(adj_q, v, w, b, bm=1024):
    n = adj_q.shape[0]
    h = v.shape[1]
    grid = (n + bm - 1) // bm

    def body(adj_ref, v_ref, w_ref, b_ref, o_ref, vq_ref, s_ref, c_ref):
        @pl.when(pl.program_id(0) == 0)
        def _():
            _quantize_to_scratch(lambda j: v_ref[j:j + _QCH, :], n, h,
                                 vq_ref, s_ref, c_ref)

        hh = jnp.maximum(_dequant_dot(adj_ref, vq_ref, s_ref, c_ref, h), 0.0)
        o_ref[...] = (
            jnp.dot(hh, w_ref[...], preferred_element_type=jnp.float32)
            + b_ref[...]
        )

    return pl.pallas_call(
        body,
        grid=(grid,),
        in_specs=[
            pl.BlockSpec((bm, n), lambda i: (i, 0)),
            pl.BlockSpec((n, h), lambda i: (0, 0)),
            pl.BlockSpec((h, h), lambda i: (0, 0)),
            pl.BlockSpec((1, h), lambda i: (0, 0)),
        ],
        out_specs=pl.BlockSpec((bm, h), lambda i: (i, 0)),
        out_shape=jax.ShapeDtypeStruct((n, h), jnp.float32),
        scratch_shapes=[
            pltpu.VMEM((n, 2 * h), _F8),
            pltpu.VMEM((1, 1), jnp.float32),
            pltpu.VMEM((1, h), jnp.float32),
        ],
        compiler_params=pltpu.CompilerParams(
            dimension_semantics=("arbitrary",)
        ),
    )(adj_q, v, w, b)


def _prop_last(adj_q, v, wp1, bp1, wp2, bp2, bm=1024):
    n = adj_q.shape[0]
    h = v.shape[1]
    p = wp1.shape[1]
    p2 = wp2.shape[1]
    grid = (n + bm - 1) // bm

    def body(adj_ref, v_ref, wp1_ref, bp1_ref, wp2_ref, bp2_ref,
             emb_ref, z_ref, vq_ref, s_ref, c_ref):
        @pl.when(pl.program_id(0) == 0)
        def _():
            _quantize_to_scratch(lambda j: v_ref[j:j + _QCH, :], n, h,
                                 vq_ref, s_ref, c_ref)

        emb = _dequant_dot(adj_ref, vq_ref, s_ref, c_ref, h)
        emb_ref[...] = emb
        t = jnp.maximum(
            jnp.dot(emb, wp1_ref[...], preferred_element_type=jnp.float32)
            + bp1_ref[...],
            0.0,
        )
        z_ref[...] = (
            jnp.dot(t, wp2_ref[...], preferred_element_type=jnp.float32)
            + bp2_ref[...]
        )

    return pl.pallas_call(
        body,
        grid=(grid,),
        in_specs=[
            pl.BlockSpec((bm, n), lambda i: (i, 0)),
            pl.BlockSpec((n, h), lambda i: (0, 0)),
            pl.BlockSpec((h, p), lambda i: (0, 0)),
            pl.BlockSpec((1, p), lambda i: (0, 0)),
            pl.BlockSpec((p, p2), lambda i: (0, 0)),
            pl.BlockSpec((1, p2), lambda i: (0, 0)),
        ],
        out_specs=[
            pl.BlockSpec((bm, h), lambda i: (i, 0)),
            pl.BlockSpec((bm, p2), lambda i: (i, 0)),
        ],
        out_shape=[
            jax.ShapeDtypeStruct((n, h), jnp.float32),
            jax.ShapeDtypeStruct((n, p2), jnp.float32),
        ],
        scratch_shapes=[
            pltpu.VMEM((n, 2 * h), _F8),
            pltpu.VMEM((1, 1), jnp.float32),
            pltpu.VMEM((1, h), jnp.float32),
        ],
        compiler_params=pltpu.CompilerParams(
            dimension_semantics=("arbitrary",)
        ),
    )(adj_q, v, wp1, bp1, wp2, bp2)


def kernel(x, Adj_, W1, b1, W2, b2, W3, b3, Wp1, bp1, Wp2, bp2):
    adj_q, v2 = _prop_first(
        Adj_, x, W1, b1.reshape(1, -1), W2, b2.reshape(1, -1)
    )
    v3 = _prop_mid(adj_q, v2, W3, b3.reshape(1, -1))
    emb, z = _prop_last(
        adj_q, v3, Wp1, bp1.reshape(1, -1), Wp2, bp2.reshape(1, -1)
    )
    return (z, emb)
